# baseline (speedup 1.0000x reference)
import functools

import jax
import jax.numpy as jnp
from jax import lax
from jax.experimental import pallas as pl
from jax.experimental.pallas import tpu as pltpu
from jax.experimental.pallas import tpu_sc as plsc

_N = 16000
_E = 512000
_NSUB = 16
_CHUNK = 1280
_EPW = _E // _NSUB
_ROWS_PW = _N // _NSUB


def _spmm_body(x_ref, init_ref, cols_ref, rows_ref, vals_ref, out_ref,
               accum, colv, rowv, valv, gbuf, sem):
    c = lax.axis_index("c")
    s = lax.axis_index("s")
    pltpu.sync_copy(init_ref.at[c].at[pl.ds(s * _ROWS_PW, _ROWS_PW)],
                    accum.at[pl.ds(s * _ROWS_PW, _ROWS_PW)])
    plsc.subcore_barrier()

    def chunk(i, carry):
        base = s * _EPW + i * _CHUNK
        pltpu.sync_copy(cols_ref.at[pl.ds(base, _CHUNK)], colv)
        pltpu.sync_copy(rows_ref.at[pl.ds(base, _CHUNK)], rowv)
        pltpu.sync_copy(vals_ref.at[pl.ds(base, _CHUNK)], valv)
        pltpu.async_copy(x_ref.at[c].at[colv], gbuf, sem).wait()

        def _blk(j, cc):
            vvec = valv[pl.ds(j * 16, 16)]
            for t in range(16):
                e = j * 16 + t
                v = vvec[t]
                gbuf[e, pl.ds(0, 16)] = gbuf[e, pl.ds(0, 16)] * v
                gbuf[e, pl.ds(16, 16)] = gbuf[e, pl.ds(16, 16)] * v
            return cc

        lax.fori_loop(0, _CHUNK // 16, _blk, 0)

        pltpu.sync_copy(gbuf, accum.at[rowv], add=True)
        return carry

    lax.fori_loop(0, _EPW // _CHUNK, chunk, 0)
    plsc.subcore_barrier()
    pltpu.sync_copy(accum.at[pl.ds(s * _ROWS_PW, _ROWS_PW)],
                    out_ref.at[c].at[pl.ds(s * _ROWS_PW, _ROWS_PW)])


def _spmm_ov_body(x_ref, init_ref, cols_ref, rows_ref, vals_ref,
                  uidx_ref, nd_ref, out_ref,
                  accum, colv, rowv, valv, gbuf, uv, ndb, sem):
    c = lax.axis_index("c")
    s = lax.axis_index("s")
    pltpu.sync_copy(init_ref.at[c].at[pl.ds(s * _ROWS_PW, _ROWS_PW)],
                    accum.at[pl.ds(s * _ROWS_PW, _ROWS_PW)])
    plsc.subcore_barrier()

    def chunk(i, carry):
        base = s * _EPW + i * _CHUNK
        pltpu.sync_copy(cols_ref.at[pl.ds(base, _CHUNK)], colv)
        pltpu.sync_copy(rows_ref.at[pl.ds(base, _CHUNK)], rowv)
        pltpu.sync_copy(vals_ref.at[pl.ds(base, _CHUNK)], valv)
        pltpu.async_copy(x_ref.at[c].at[colv], gbuf, sem).wait()

        def _blk(j, cc):
            vvec = valv[pl.ds(j * 16, 16)]
            for t in range(16):
                e = j * 16 + t
                v = vvec[t]
                gbuf[e, pl.ds(0, 16)] = gbuf[e, pl.ds(0, 16)] * v
                gbuf[e, pl.ds(16, 16)] = gbuf[e, pl.ds(16, 16)] * v
            return cc

        lax.fori_loop(0, _CHUNK // 16, _blk, 0)
        pltpu.sync_copy(gbuf, accum.at[rowv], add=True)
        return carry

    lax.fori_loop(0, _EPW // _CHUNK, chunk, 0)
    plsc.subcore_barrier()

    pltpu.sync_copy(accum.at[pl.ds(s * _ROWS_PW, _ROWS_PW)],
                    out_ref.at[c].at[pl.ds(s * _ROWS_PW, _ROWS_PW)])
    plsc.subcore_barrier()

    @pl.when(s == 0)
    def _overwrite():
        pltpu.sync_copy(uidx_ref, uv)
        pltpu.sync_copy(nd_ref.at[c], ndb)
        pltpu.sync_copy(ndb, out_ref.at[c].at[uv])


@jax.jit
def _spmm_ov_sc(x_split, init_split, cols, rows, vals, uidx, nd):
    mesh = plsc.VectorSubcoreMesh(core_axis_name="c", subcore_axis_name="s",
                                  num_cores=2, num_subcores=_NSUB)
    return pl.kernel(
        _spmm_ov_body,
        out_type=jax.ShapeDtypeStruct((2, _N, 32), jnp.float32),
        mesh=mesh,
        compiler_params=pltpu.CompilerParams(use_tc_tiling_on_sc=False, needs_layout_passes=False),
        scratch_types=[
            pltpu.VMEM_SHARED((_N, 32), jnp.float32),
            pltpu.VMEM((_CHUNK,), jnp.int32),
            pltpu.VMEM((_CHUNK,), jnp.int32),
            pltpu.VMEM((_CHUNK,), jnp.float32),
            pltpu.VMEM((_CHUNK, 32), jnp.float32),
            pltpu.VMEM((256,), jnp.int32),
            pltpu.VMEM((256, 32), jnp.float32),
            pltpu.SemaphoreType.DMA,
        ],
    )(x_split, init_split, cols, rows, vals, uidx, nd)


@jax.jit
def _spmm_sc(x_split, init_split, cols, rows, vals):
    mesh = plsc.VectorSubcoreMesh(core_axis_name="c", subcore_axis_name="s",
                                  num_cores=2, num_subcores=_NSUB)
    return pl.kernel(
        _spmm_body,
        out_type=jax.ShapeDtypeStruct((2, _N, 32), jnp.float32),
        mesh=mesh,
        compiler_params=pltpu.CompilerParams(use_tc_tiling_on_sc=False, needs_layout_passes=False),
        scratch_types=[
            pltpu.VMEM_SHARED((_N, 32), jnp.float32),
            pltpu.VMEM((_CHUNK,), jnp.int32),
            pltpu.VMEM((_CHUNK,), jnp.int32),
            pltpu.VMEM((_CHUNK,), jnp.float32),
            pltpu.VMEM((_CHUNK, 32), jnp.float32),
            pltpu.SemaphoreType.DMA,
        ],
    )(x_split, init_split, cols, rows, vals)


def _to_split(x):
    return x.reshape(_N, 2, 32).transpose(1, 0, 2)


def _from_split(xs):
    return xs.transpose(1, 0, 2).reshape(_N, 64)


_CAP = 32768
_FCH = 1600
_MCH = 256


def _fill_i32(ref, n, value):
    def w(j, cc):
        ref[pl.ds(j * 16, 16)] = jnp.full((16,), value, jnp.int32)
        return cc
    lax.fori_loop(0, n // 16, w, 0)


def _filter_body(rows_ref, cols_ref, vals_ref, uidx_ref, ar256_ref,
                 inv_ref, segA_ref, segB_ref, segV_ref, cnt_ref,
                 invb, uv, arv, keyb, othb, valb,
                 cbufA, cbufB, cbufV, cntb, sem):
    c = lax.axis_index("c")
    s = lax.axis_index("s")
    npw = _N // _NSUB
    _fill_i32(invb, _N, -1)
    pltpu.sync_copy(uidx_ref, uv)
    pltpu.sync_copy(ar256_ref, arv)
    for j in range(16):
        u16 = uv[pl.ds(j * 16, 16)]
        a16 = arv[pl.ds(j * 16, 16)]
        plsc.store_scatter(invb, [u16], a16)

    @pl.when(c == 0)
    def _inv_out():
        pltpu.sync_copy(invb.at[pl.ds(s * npw, npw)],
                        inv_ref.at[pl.ds(s * npw, npw)])

    epw = _E // _NSUB

    def chunk(i, cnt):
        base = s * epw + i * _FCH

        @pl.when(c == 0)
        def _stage_r():
            pltpu.sync_copy(rows_ref.at[pl.ds(base, _FCH)], keyb)
            pltpu.sync_copy(cols_ref.at[pl.ds(base, _FCH)], othb)

        @pl.when(c == 1)
        def _stage_c():
            pltpu.sync_copy(cols_ref.at[pl.ds(base, _FCH)], keyb)
            pltpu.sync_copy(rows_ref.at[pl.ds(base, _FCH)], othb)

        pltpu.sync_copy(vals_ref.at[pl.ds(base, _FCH)], valb)

        def blk(j, cnt):
            key = keyb[pl.ds(j * 16, 16)]
            sl = plsc.load_gather(invb, [key])
            ot = othb[pl.ds(j * 16, 16)]
            vv = valb[pl.ds(j * 16, 16)]
            m = sl >= 0
            mi = jnp.where(m, jnp.full((16,), 1, jnp.int32), jnp.full((16,), 0, jnp.int32))
            csum = plsc.cumsum(mi)
            pos = jnp.where(m, cnt + csum - mi, _CAP)
            plsc.store_scatter(cbufA, [pos], sl)
            plsc.store_scatter(cbufB, [pos], ot)
            plsc.store_scatter(cbufV, [pos], vv)
            return cnt + csum[15]

        return lax.fori_loop(0, _FCH // 16, blk, cnt)

    cnt = lax.fori_loop(0, epw // _FCH, chunk, jnp.int32(0))
    pltpu.sync_copy(cbufA.at[pl.ds(0, _CAP)], segA_ref.at[c].at[s])
    pltpu.sync_copy(cbufB.at[pl.ds(0, _CAP)], segB_ref.at[c].at[s])
    pltpu.sync_copy(cbufV.at[pl.ds(0, _CAP)], segV_ref.at[c].at[s])
    cntb[pl.ds(0, 16)] = jnp.full((16,), cnt, jnp.int32)
    pltpu.sync_copy(cntb, cnt_ref.at[c].at[s])


@jax.jit
def _filter_sc(rows, cols, vals, uidx, ar256):
    mesh = plsc.VectorSubcoreMesh(core_axis_name="c", subcore_axis_name="s",
                                  num_cores=2, num_subcores=_NSUB)
    return pl.kernel(
        _filter_body,
        out_type=(jax.ShapeDtypeStruct((_N,), jnp.int32),
                  jax.ShapeDtypeStruct((2, _NSUB, _CAP), jnp.int32),
                  jax.ShapeDtypeStruct((2, _NSUB, _CAP), jnp.int32),
                  jax.ShapeDtypeStruct((2, _NSUB, _CAP), jnp.float32),
                  jax.ShapeDtypeStruct((2, _NSUB, 16), jnp.int32)),
        mesh=mesh,
        compiler_params=pltpu.CompilerParams(use_tc_tiling_on_sc=False, needs_layout_passes=False),
        scratch_types=[
            pltpu.VMEM((_N,), jnp.int32),
            pltpu.VMEM((256,), jnp.int32),
            pltpu.VMEM((256,), jnp.int32),
            pltpu.VMEM((_FCH,), jnp.int32),
            pltpu.VMEM((_FCH,), jnp.int32),
            pltpu.VMEM((_FCH,), jnp.float32),
            pltpu.VMEM((_CAP + 16,), jnp.int32),
            pltpu.VMEM((_CAP + 16,), jnp.int32),
            pltpu.VMEM((_CAP + 16,), jnp.float32),
            pltpu.VMEM((16,), jnp.int32),
            pltpu.SemaphoreType.DMA,
        ],
    )(rows, cols, vals, uidx, ar256)


def _newden_body(h_ref, d_ref, segA_ref, segB_ref, segV_ref, cnt_ref,
                 uidx_ref, ar256_ref, nd_ref, hu_ref,
                 accS0, accS1,
                 uv, arv, hu0, hu1, accL0, accL1,
                 slotb, colb, valb, gh0, gh1, gd0, gd1, cntb, sem):
    s = lax.axis_index("s")

    def zr(j, cc):
        accL0[j, pl.ds(0, 16)] = jnp.zeros((16,), jnp.float32)
        accL0[j, pl.ds(16, 16)] = jnp.zeros((16,), jnp.float32)
        accL1[j, pl.ds(0, 16)] = jnp.zeros((16,), jnp.float32)
        accL1[j, pl.ds(16, 16)] = jnp.zeros((16,), jnp.float32)
        return cc
    lax.fori_loop(0, 256, zr, 0)

    @pl.when(s == 0)
    def _zero_shared():
        pltpu.sync_copy(accL0, accS0)
        pltpu.sync_copy(accL1, accS1)

    pltpu.sync_copy(ar256_ref, arv)
    pltpu.sync_copy(uidx_ref, uv)
    pltpu.async_copy(h_ref.at[0].at[uv], hu0, sem).wait()
    pltpu.async_copy(h_ref.at[1].at[uv], hu1, sem).wait()

    pltpu.sync_copy(cnt_ref.at[0].at[s], cntb)
    n = cntb[pl.ds(0, 16)][0]
    plsc.subcore_barrier()

    def chunk(i, cc):
        base = i * _MCH
        pltpu.sync_copy(segA_ref.at[0].at[s].at[pl.ds(base, _MCH)], slotb)
        pltpu.sync_copy(segB_ref.at[0].at[s].at[pl.ds(base, _MCH)], colb)
        pltpu.sync_copy(segV_ref.at[0].at[s].at[pl.ds(base, _MCH)], valb)

        def san(j, cc2):
            e16 = base + j * 16 + lax.iota(jnp.int32, 16)
            live = e16 < n
            slotb[pl.ds(j * 16, 16)] = jnp.where(live, slotb[pl.ds(j * 16, 16)], 0)
            colb[pl.ds(j * 16, 16)] = jnp.where(live, colb[pl.ds(j * 16, 16)], 0)
            valb[pl.ds(j * 16, 16)] = jnp.where(live, valb[pl.ds(j * 16, 16)], 0.0)
            return cc2
        lax.fori_loop(0, _MCH // 16, san, 0)

        pltpu.async_copy(h_ref.at[0].at[colb], gh0, sem).wait()
        pltpu.async_copy(h_ref.at[1].at[colb], gh1, sem).wait()
        pltpu.async_copy(d_ref.at[0].at[colb], gd0, sem).wait()
        pltpu.async_copy(d_ref.at[1].at[colb], gd1, sem).wait()

        def blk(j, cc2):
            sl16 = slotb[pl.ds(j * 16, 16)]
            vv16 = valb[pl.ds(j * 16, 16)]
            for t in range(16):
                e = j * 16 + t
                p = sl16[t]
                v = vv16[t]
                part = (gh0[e, pl.ds(0, 16)] * hu0[p, pl.ds(0, 16)]
                        + gh0[e, pl.ds(16, 16)] * hu0[p, pl.ds(16, 16)]
                        + gh1[e, pl.ds(0, 16)] * hu1[p, pl.ds(0, 16)]
                        + gh1[e, pl.ds(16, 16)] * hu1[p, pl.ds(16, 16)])
                dot = jnp.sum(part)
                sigv = 1.0 / (1.0 + jnp.exp(jnp.full((16,), -dot, jnp.float32)))
                wv = sigv * v
                accL0[p, pl.ds(0, 16)] = accL0[p, pl.ds(0, 16)] + wv * gd0[e, pl.ds(0, 16)]
                accL0[p, pl.ds(16, 16)] = accL0[p, pl.ds(16, 16)] + wv * gd0[e, pl.ds(16, 16)]
                accL1[p, pl.ds(0, 16)] = accL1[p, pl.ds(0, 16)] + wv * gd1[e, pl.ds(0, 16)]
                accL1[p, pl.ds(16, 16)] = accL1[p, pl.ds(16, 16)] + wv * gd1[e, pl.ds(16, 16)]
            return cc2
        lax.fori_loop(0, _MCH // 16, blk, 0)
        return cc

    nch = lax.div(n + (_MCH - 1), _MCH)
    lax.fori_loop(0, nch, chunk, 0)
    pltpu.sync_copy(accL0, accS0.at[arv], add=True)
    pltpu.sync_copy(accL1, accS1.at[arv], add=True)
    plsc.subcore_barrier()

    @pl.when(s == 0)
    def _writeout():
        pltpu.sync_copy(accS0, nd_ref.at[0])
        pltpu.sync_copy(accS1, nd_ref.at[1])
        pltpu.sync_copy(hu0, hu_ref.at[0])
        pltpu.sync_copy(hu1, hu_ref.at[1])


@jax.jit
def _newden_sc(h_split, d_split, segA, segB, segV, counts, uidx, ar256):
    mesh = plsc.VectorSubcoreMesh(core_axis_name="c", subcore_axis_name="s",
                                  num_cores=2, num_subcores=_NSUB)
    return pl.kernel(
        _newden_body,
        out_type=(jax.ShapeDtypeStruct((2, 256, 32), jnp.float32),
                  jax.ShapeDtypeStruct((2, 256, 32), jnp.float32)),
        mesh=mesh,
        compiler_params=pltpu.CompilerParams(use_tc_tiling_on_sc=False, needs_layout_passes=False),
        scratch_types=[
            pltpu.VMEM_SHARED((256, 32), jnp.float32),
            pltpu.VMEM_SHARED((256, 32), jnp.float32),
            pltpu.VMEM((256,), jnp.int32),
            pltpu.VMEM((256,), jnp.int32),
            pltpu.VMEM((256, 32), jnp.float32),
            pltpu.VMEM((256, 32), jnp.float32),
            pltpu.VMEM((256, 32), jnp.float32),
            pltpu.VMEM((256, 32), jnp.float32),
            pltpu.VMEM((_MCH,), jnp.int32),
            pltpu.VMEM((_MCH,), jnp.int32),
            pltpu.VMEM((_MCH,), jnp.float32),
            pltpu.VMEM((_MCH, 32), jnp.float32),
            pltpu.VMEM((_MCH, 32), jnp.float32),
            pltpu.VMEM((_MCH, 32), jnp.float32),
            pltpu.VMEM((_MCH, 32), jnp.float32),
            pltpu.VMEM((16,), jnp.int32),
            pltpu.SemaphoreType.DMA,
        ],
    )(h_split, d_split, segA, segB, segV, counts, uidx, ar256)


def _dspmm_body(base2_ref, segA_ref, segB_ref, segV_ref, cnt_ref,
                nd0_ref, e0_ref, e1_ref, uidx_ref, out_ref,
                accum, uv, deltab, t0b, t1b, rowb, slotb, valb, contrib,
                cntb, sem):
    c = lax.axis_index("c")
    s = lax.axis_index("s")
    npw = _N // _NSUB
    pltpu.sync_copy(base2_ref.at[c].at[pl.ds(s * npw, npw)],
                    accum.at[pl.ds(s * npw, npw)])

    pltpu.sync_copy(uidx_ref, uv)
    pltpu.async_copy(e0_ref.at[c].at[uv], t0b, sem).wait()
    pltpu.async_copy(e1_ref.at[c].at[uv], t1b, sem).wait()
    pltpu.sync_copy(nd0_ref.at[c], deltab)

    def dl(j, cc):
        deltab[j, pl.ds(0, 16)] = deltab[j, pl.ds(0, 16)] - (t1b[j, pl.ds(0, 16)] - t0b[j, pl.ds(0, 16)])
        deltab[j, pl.ds(16, 16)] = deltab[j, pl.ds(16, 16)] - (t1b[j, pl.ds(16, 16)] - t0b[j, pl.ds(16, 16)])
        return cc
    lax.fori_loop(0, 256, dl, 0)

    pltpu.sync_copy(cnt_ref.at[1].at[s], cntb)
    n = cntb[pl.ds(0, 16)][0]
    plsc.subcore_barrier()

    def chunk(i, cc):
      base = i * _MCH

      @pl.when(base < n)
      def _live_chunk():
        pltpu.sync_copy(segA_ref.at[1].at[s].at[pl.ds(base, _MCH)], slotb)
        pltpu.sync_copy(segB_ref.at[1].at[s].at[pl.ds(base, _MCH)], rowb)
        pltpu.sync_copy(segV_ref.at[1].at[s].at[pl.ds(base, _MCH)], valb)

        def san(j, cc2):
            e16 = base + j * 16 + lax.iota(jnp.int32, 16)
            live = e16 < n
            slotb[pl.ds(j * 16, 16)] = jnp.where(live, slotb[pl.ds(j * 16, 16)], 0)
            rowb[pl.ds(j * 16, 16)] = jnp.where(live, rowb[pl.ds(j * 16, 16)], 0)
            valb[pl.ds(j * 16, 16)] = jnp.where(live, valb[pl.ds(j * 16, 16)], 0.0)
            return cc2
        lax.fori_loop(0, _MCH // 16, san, 0)

        def blk(j, cc2):
            sl16 = slotb[pl.ds(j * 16, 16)]
            vv16 = valb[pl.ds(j * 16, 16)]
            for t in range(16):
                e = j * 16 + t
                p = sl16[t]
                v = vv16[t]
                contrib[e, pl.ds(0, 16)] = v * deltab[p, pl.ds(0, 16)]
                contrib[e, pl.ds(16, 16)] = v * deltab[p, pl.ds(16, 16)]
            return cc2
        lax.fori_loop(0, _MCH // 16, blk, 0)
        pltpu.sync_copy(contrib, accum.at[rowb], add=True)
      return cc

    lax.fori_loop(0, _CAP // _MCH, chunk, 0)
    plsc.subcore_barrier()

    pltpu.sync_copy(accum.at[pl.ds(s * npw, npw)],
                    out_ref.at[c].at[pl.ds(s * npw, npw)])


@jax.jit
def _dspmm_sc(base2s, segA, segB, segV, counts, nd0, e0s, e1s, uidx):
    mesh = plsc.VectorSubcoreMesh(core_axis_name="c", subcore_axis_name="s",
                                  num_cores=2, num_subcores=_NSUB)
    return pl.kernel(
        _dspmm_body,
        out_type=jax.ShapeDtypeStruct((2, _N, 32), jnp.float32),
        mesh=mesh,
        compiler_params=pltpu.CompilerParams(use_tc_tiling_on_sc=False, needs_layout_passes=False),
        scratch_types=[
            pltpu.VMEM_SHARED((_N, 32), jnp.float32),
            pltpu.VMEM((256,), jnp.int32),
            pltpu.VMEM((256, 32), jnp.float32),
            pltpu.VMEM((256, 32), jnp.float32),
            pltpu.VMEM((256, 32), jnp.float32),
            pltpu.VMEM((_MCH,), jnp.int32),
            pltpu.VMEM((_MCH,), jnp.int32),
            pltpu.VMEM((_MCH,), jnp.float32),
            pltpu.VMEM((_MCH, 32), jnp.float32),
            pltpu.VMEM((16,), jnp.int32),
            pltpu.SemaphoreType.DMA,
        ],
    )(base2s, segA, segB, segV, counts, nd0, e0s, e1s, uidx)


def _final_mix_kernel(e0_ref, e1_ref, e2_ref, e3_ref, noise_ref, sh_ref,
                      wm_ref, ws_ref, wd_ref,
                      main_ref, mean_ref, std_ref, dec_ref,
                      h0_ref, h1_ref, h2_ref, b2_ref, t0_ref, s_ref):
    i = pl.program_id(0)
    e0 = e0_ref[...]
    e1 = e1_ref[...]
    e2 = e2_ref[...]
    e3 = e3_ref[...]
    m = (e0 + e1 + e2 + e3) * 0.25
    main_ref[...] = m
    mean = jnp.maximum(m @ wm_ref[...], 0.0)
    std = jnp.maximum(m @ ws_ref[...], 0.0)
    mean_ref[...] = mean
    std_ref[...] = std
    x = noise_ref[...] * std + mean
    dec = jnp.maximum(x @ wd_ref[...], 0.0)
    dec_ref[...] = dec
    sh = sh_ref[...]
    h0_ref[...] = jax.nn.sigmoid(sh * e0)
    h1_ref[...] = jax.nn.sigmoid(sh * e1)
    h2_ref[...] = jax.nn.sigmoid(sh * e2)
    b2_ref[...] = e2 - 2.0 * e1 + e0

    @pl.when(i == 0)
    def _init():
        t0_ref[...] = jnp.zeros((64, 64), jnp.float32)
        s_ref[...] = jnp.zeros((64, 64), jnp.float32)

    t0_ref[...] += dec.T @ e0
    s_ref[...] += dec.T @ dec


def _dense_pass(e0, e1, e2, e3, noise, shared, W_mean, W_std, W_decoder):
    n = e0.shape[0]
    blk = 3200
    grid = (n // blk,)
    bs = pl.BlockSpec((blk, 64), lambda i: (i, 0))
    ws = pl.BlockSpec((64, 64), lambda i: (0, 0))
    shs = pl.BlockSpec((1, 64), lambda i: (0, 0))
    acc = pl.BlockSpec((64, 64), lambda i: (0, 0))
    out_sd = jax.ShapeDtypeStruct((n, 64), jnp.float32)
    w_sd = jax.ShapeDtypeStruct((64, 64), jnp.float32)
    return pl.pallas_call(
        _final_mix_kernel,
        grid=grid,
        in_specs=[bs, bs, bs, bs, bs, shs, ws, ws, ws],
        out_specs=[bs, bs, bs, bs, bs, bs, bs, bs, acc, acc],
        out_shape=[out_sd] * 8 + [w_sd, w_sd],
    )(e0, e1, e2, e3, noise, shared, W_mean, W_std, W_decoder)


def _m_kernel(t0_ref, s_ref, m_ref):
    t0 = t0_ref[...]
    s = s_ref[...]
    st = s @ t0
    m_ref[...] = t0 + st + s @ st


def _gen_m_pass(t0, S):
    return pl.pallas_call(
        _m_kernel,
        out_shape=jax.ShapeDtypeStruct((64, 64), jnp.float32),
    )(t0, S)


def _ov_kernel(base_ref, nd_ref, inv_ref, out_ref):
    inv = inv_ref[...]
    flag = inv >= 0
    oh = (inv == lax.broadcasted_iota(jnp.int32, (1, 256), 1)).astype(jnp.float32)
    out_ref[...] = jnp.where(flag, oh @ nd_ref[...], base_ref[...])


def _ov_pass(base, nd, inv2d):
    n = base.shape[0]
    blk = 3200
    bs = pl.BlockSpec((blk, 64), lambda i: (i, 0))
    return pl.pallas_call(
        _ov_kernel,
        grid=(n // blk,),
        in_specs=[bs, pl.BlockSpec((256, 64), lambda i: (0, 0)),
                  pl.BlockSpec((blk, 1), lambda i: (i, 0))],
        out_specs=bs,
        out_shape=jax.ShapeDtypeStruct((n, 64), jnp.float32),
    )(base, nd, inv2d)


def _fin_kernel(e0_ref, d1_ref, d2_ref, d3r_ref, nd2_ref, inv_ref,
                dec_ref, m_ref, h0_ref, h1_ref, h2_ref,
                hu0_ref, hu1_ref, hu2_ref,
                den_ref, gen_ref, dg0_ref, dg1_ref, dg2_ref):
    inv = inv_ref[...]
    flag = inv >= 0
    oh = (inv == lax.broadcasted_iota(jnp.int32, (1, 256), 1)).astype(jnp.float32)
    d3 = jnp.where(flag, oh @ nd2_ref[...], d3r_ref[...])
    e0 = e0_ref[...]
    den_ref[...] = (e0 + d1_ref[...] + d2_ref[...] + d3) * 0.25
    gen_ref[...] = (e0 + dec_ref[...] @ m_ref[...]) * 0.25
    dg0_ref[...] = jax.nn.sigmoid(hu0_ref[...] @ h0_ref[...].T)
    dg1_ref[...] = jax.nn.sigmoid(hu1_ref[...] @ h1_ref[...].T)
    dg2_ref[...] = jax.nn.sigmoid(hu2_ref[...] @ h2_ref[...].T)


def _final_pass(e0, d1, d2, d3raw, nd2, inv2d, dec, M, h0, h1, h2,
                hu0, hu1, hu2):
    n = e0.shape[0]
    blk = 3200
    bs = pl.BlockSpec((blk, 64), lambda i: (i, 0))
    hus = pl.BlockSpec((256, 64), lambda i: (0, 0))
    ms = pl.BlockSpec((64, 64), lambda i: (0, 0))
    ivs = pl.BlockSpec((blk, 1), lambda i: (i, 0))
    dgs = pl.BlockSpec((256, blk), lambda i: (0, i))
    out_sd = jax.ShapeDtypeStruct((n, 64), jnp.float32)
    dg_sd = jax.ShapeDtypeStruct((256, n), jnp.float32)
    return pl.pallas_call(
        _fin_kernel,
        grid=(n // blk,),
        in_specs=[bs, bs, bs, bs, hus, ivs, bs, ms, bs, bs, bs,
                  hus, hus, hus],
        out_specs=[bs, bs, dgs, dgs, dgs],
        out_shape=[out_sd, out_sd, dg_sd, dg_sd, dg_sd],
    )(e0, d1, d2, d3raw, nd2, inv2d, dec, M, h0, h1, h2, hu0, hu1, hu2)


def kernel(user_emb, item_emb, W_mean, W_std, W_decoder, shared_layer,
           adj_rows, adj_cols, adj_vals, user_idx, noise):
    user_num = user_emb.shape[0]
    e0 = jnp.concatenate([user_emb, item_emb], axis=0)
    n_total = e0.shape[0]

    zeros_split = jnp.zeros((2, n_total, 32), jnp.float32)
    e0s = _to_split(e0)
    e1s = _spmm_sc(e0s, e0s, adj_cols, adj_rows, adj_vals)
    e2s = _spmm_sc(e1s, e1s, adj_cols, adj_rows, adj_vals)
    e3s = _spmm_sc(e2s, e2s, adj_cols, adj_rows, adj_vals)
    e1 = _from_split(e1s)
    e2 = _from_split(e2s)
    e3 = _from_split(e3s)

    (main_all, mean, std, D, h0, h1, h2, base2, t0, S) = _dense_pass(
        e0, e1, e2, e3, noise, shared_layer, W_mean, W_std, W_decoder)
    M = _gen_m_pass(t0, S)

    ar256 = jnp.arange(256, dtype=jnp.int32)
    inv, segA, segB, segV, counts = _filter_sc(adj_rows, adj_cols, adj_vals,
                                               user_idx, ar256)
    inv2d = inv[:, None]
    slot_of = inv[user_idx]

    def _dup(nds):
        full = jnp.concatenate([nds[0], nds[1]], axis=1)[slot_of]
        return full, jnp.stack([full[:, :32], full[:, 32:]], axis=0)

    hs = [_to_split(hk) for hk in (h0, h1, h2)]

    nd0r, hu0 = _newden_sc(hs[0], e0s, segA, segB, segV, counts, user_idx, ar256)
    nd0, nd0s = _dup(nd0r)
    d1 = _ov_pass(e1 - e0, nd0, inv2d)

    d1s = _to_split(d1)
    nd1r, hu1 = _newden_sc(hs[1], d1s, segA, segB, segV, counts, user_idx, ar256)
    nd1, _ = _dup(nd1r)
    d2raw = _from_split(_dspmm_sc(_to_split(base2), segA, segB, segV, counts,
                                  nd0s, e0s, e1s, user_idx))
    d2 = _ov_pass(d2raw, nd1, inv2d)
    d2s = _to_split(d2)
    nd2r, hu2 = _newden_sc(hs[2], d2s, segA, segB, segV, counts, user_idx, ar256)
    nd2, _ = _dup(nd2r)
    d3raw = _from_split(_spmm_sc(d2s, zeros_split, adj_cols, adj_rows, adj_vals))

    def _huf(hu):
        return jnp.concatenate([hu[0], hu[1]], axis=1)

    den_m, gen_m, dg0, dg1, dg2 = _final_pass(
        e0, d1, d2, d3raw, nd2, inv2d, D, M, h0, h1, h2,
        _huf(hu0), _huf(hu1), _huf(hu2))

    return (main_all[:user_num], main_all[user_num:],
            gen_m[:user_num], gen_m[user_num:],
            den_m[:user_num], den_m[user_num:],
            mean, std, (dg0, dg1, dg2))


# speedup vs baseline: 1.0705x; 1.0705x over previous
import functools

import jax
import jax.numpy as jnp
from jax import lax
from jax.experimental import pallas as pl
from jax.experimental.pallas import tpu as pltpu
from jax.experimental.pallas import tpu_sc as plsc

_N = 16000
_E = 512000
_NSUB = 16
_CHUNK = 800
_EPW = _E // _NSUB
_ROWS_PW = _N // _NSUB


_NCH = _EPW // _CHUNK
_NPAIR = _NCH // 2


def _scale_chunk(gbuf, valv):
    def _blk(j, cc):
        vvec = valv[pl.ds(j * 16, 16)]
        for t in range(16):
            e = j * 16 + t
            v = vvec[t]
            gbuf[e, pl.ds(0, 16)] = gbuf[e, pl.ds(0, 16)] * v
            gbuf[e, pl.ds(16, 16)] = gbuf[e, pl.ds(16, 16)] * v
        return cc
    lax.fori_loop(0, _CHUNK // 16, _blk, 0)


def _spmm_body(x_ref, init_ref, cols_ref, rows_ref, vals_ref, out_ref,
               accum, colvA, rowvA, valvA, gbufA,
               colvB, rowvB, valvB, gbufB, semA, semB):
    c = lax.axis_index("c")
    s = lax.axis_index("s")
    pltpu.sync_copy(init_ref.at[c].at[pl.ds(s * _ROWS_PW, _ROWS_PW)],
                    accum.at[pl.ds(s * _ROWS_PW, _ROWS_PW)])
    plsc.subcore_barrier()

    def stage(base, colv, rowv, valv):
        pltpu.sync_copy(cols_ref.at[pl.ds(base, _CHUNK)], colv)
        pltpu.sync_copy(rows_ref.at[pl.ds(base, _CHUNK)], rowv)
        pltpu.sync_copy(vals_ref.at[pl.ds(base, _CHUNK)], valv)

    stage(s * _EPW, colvA, rowvA, valvA)
    pltpu.async_copy(x_ref.at[c].at[colvA], gbufA, semA)

    def pair(j, carry):
        baseB = s * _EPW + (2 * j + 1) * _CHUNK
        stage(baseB, colvB, rowvB, valvB)
        dB = pltpu.async_copy(x_ref.at[c].at[colvB], gbufB, semB)

        pltpu.make_async_copy(x_ref.at[c].at[colvA], gbufA, semA).wait()
        _scale_chunk(gbufA, valvA)
        pltpu.sync_copy(gbufA, accum.at[rowvA], add=True)

        @pl.when(j + 1 < _NPAIR)
        def _prefetch_a():
            baseA = s * _EPW + (2 * j + 2) * _CHUNK
            stage(baseA, colvA, rowvA, valvA)
            pltpu.async_copy(x_ref.at[c].at[colvA], gbufA, semA)

        dB.wait()
        _scale_chunk(gbufB, valvB)
        pltpu.sync_copy(gbufB, accum.at[rowvB], add=True)
        return carry

    lax.fori_loop(0, _NPAIR, pair, 0)
    plsc.subcore_barrier()
    pltpu.sync_copy(accum.at[pl.ds(s * _ROWS_PW, _ROWS_PW)],
                    out_ref.at[c].at[pl.ds(s * _ROWS_PW, _ROWS_PW)])


@jax.jit
def _spmm_sc(x_split, init_split, cols, rows, vals):
    mesh = plsc.VectorSubcoreMesh(core_axis_name="c", subcore_axis_name="s",
                                  num_cores=2, num_subcores=_NSUB)
    return pl.kernel(
        _spmm_body,
        out_type=jax.ShapeDtypeStruct((2, _N, 32), jnp.float32),
        mesh=mesh,
        compiler_params=pltpu.CompilerParams(use_tc_tiling_on_sc=False, needs_layout_passes=False),
        scratch_types=[
            pltpu.VMEM_SHARED((_N, 32), jnp.float32),
            pltpu.VMEM((_CHUNK,), jnp.int32),
            pltpu.VMEM((_CHUNK,), jnp.int32),
            pltpu.VMEM((_CHUNK,), jnp.float32),
            pltpu.VMEM((_CHUNK, 32), jnp.float32),
            pltpu.VMEM((_CHUNK,), jnp.int32),
            pltpu.VMEM((_CHUNK,), jnp.int32),
            pltpu.VMEM((_CHUNK,), jnp.float32),
            pltpu.VMEM((_CHUNK, 32), jnp.float32),
            pltpu.SemaphoreType.DMA,
            pltpu.SemaphoreType.DMA,
        ],
    )(x_split, init_split, cols, rows, vals)


def _spmm_ov_body(x_ref, init_ref, cols_ref, rows_ref, vals_ref,
                  uidx_ref, nd_ref, out_ref,
                  accum, colv, rowv, valv, gbuf, uv, ndb, sem):
    c = lax.axis_index("c")
    s = lax.axis_index("s")
    pltpu.sync_copy(init_ref.at[c].at[pl.ds(s * _ROWS_PW, _ROWS_PW)],
                    accum.at[pl.ds(s * _ROWS_PW, _ROWS_PW)])
    plsc.subcore_barrier()

    def chunk(i, carry):
        base = s * _EPW + i * _CHUNK
        pltpu.sync_copy(cols_ref.at[pl.ds(base, _CHUNK)], colv)
        pltpu.sync_copy(rows_ref.at[pl.ds(base, _CHUNK)], rowv)
        pltpu.sync_copy(vals_ref.at[pl.ds(base, _CHUNK)], valv)
        pltpu.async_copy(x_ref.at[c].at[colv], gbuf, sem).wait()

        def _blk(j, cc):
            vvec = valv[pl.ds(j * 16, 16)]
            for t in range(16):
                e = j * 16 + t
                v = vvec[t]
                gbuf[e, pl.ds(0, 16)] = gbuf[e, pl.ds(0, 16)] * v
                gbuf[e, pl.ds(16, 16)] = gbuf[e, pl.ds(16, 16)] * v
            return cc

        lax.fori_loop(0, _CHUNK // 16, _blk, 0)
        pltpu.sync_copy(gbuf, accum.at[rowv], add=True)
        return carry

    lax.fori_loop(0, _EPW // _CHUNK, chunk, 0)
    plsc.subcore_barrier()

    pltpu.sync_copy(accum.at[pl.ds(s * _ROWS_PW, _ROWS_PW)],
                    out_ref.at[c].at[pl.ds(s * _ROWS_PW, _ROWS_PW)])
    plsc.subcore_barrier()

    @pl.when(s == 0)
    def _overwrite():
        pltpu.sync_copy(uidx_ref, uv)
        pltpu.sync_copy(nd_ref.at[c], ndb)
        pltpu.sync_copy(ndb, out_ref.at[c].at[uv])


@jax.jit
def _spmm_ov_sc(x_split, init_split, cols, rows, vals, uidx, nd):
    mesh = plsc.VectorSubcoreMesh(core_axis_name="c", subcore_axis_name="s",
                                  num_cores=2, num_subcores=_NSUB)
    return pl.kernel(
        _spmm_ov_body,
        out_type=jax.ShapeDtypeStruct((2, _N, 32), jnp.float32),
        mesh=mesh,
        compiler_params=pltpu.CompilerParams(use_tc_tiling_on_sc=False, needs_layout_passes=False),
        scratch_types=[
            pltpu.VMEM_SHARED((_N, 32), jnp.float32),
            pltpu.VMEM((_CHUNK,), jnp.int32),
            pltpu.VMEM((_CHUNK,), jnp.int32),
            pltpu.VMEM((_CHUNK,), jnp.float32),
            pltpu.VMEM((_CHUNK, 32), jnp.float32),
            pltpu.VMEM((256,), jnp.int32),
            pltpu.VMEM((256, 32), jnp.float32),
            pltpu.SemaphoreType.DMA,
        ],
    )(x_split, init_split, cols, rows, vals, uidx, nd)


def _to_split(x):
    return x.reshape(_N, 2, 32).transpose(1, 0, 2)


def _from_split(xs):
    return xs.transpose(1, 0, 2).reshape(_N, 64)


_CAP = 32768
_FCH = 1600
_MCH = 256


def _fill_i32(ref, n, value):
    def w(j, cc):
        ref[pl.ds(j * 16, 16)] = jnp.full((16,), value, jnp.int32)
        return cc
    lax.fori_loop(0, n // 16, w, 0)


def _filter_body(rows_ref, cols_ref, vals_ref, uidx_ref, ar256_ref,
                 inv_ref, segA_ref, segB_ref, segV_ref, cnt_ref,
                 invb, uv, arv, keyb, othb, valb,
                 cbufA, cbufB, cbufV, cntb, sem):
    c = lax.axis_index("c")
    s = lax.axis_index("s")
    npw = _N // _NSUB
    _fill_i32(invb, _N, -1)
    pltpu.sync_copy(uidx_ref, uv)
    pltpu.sync_copy(ar256_ref, arv)
    for j in range(16):
        u16 = uv[pl.ds(j * 16, 16)]
        a16 = arv[pl.ds(j * 16, 16)]
        plsc.store_scatter(invb, [u16], a16)

    @pl.when(c == 0)
    def _inv_out():
        pltpu.sync_copy(invb.at[pl.ds(s * npw, npw)],
                        inv_ref.at[pl.ds(s * npw, npw)])

    epw = _E // _NSUB

    def chunk(i, cnt):
        base = s * epw + i * _FCH

        @pl.when(c == 0)
        def _stage_r():
            pltpu.sync_copy(rows_ref.at[pl.ds(base, _FCH)], keyb)
            pltpu.sync_copy(cols_ref.at[pl.ds(base, _FCH)], othb)

        @pl.when(c == 1)
        def _stage_c():
            pltpu.sync_copy(cols_ref.at[pl.ds(base, _FCH)], keyb)
            pltpu.sync_copy(rows_ref.at[pl.ds(base, _FCH)], othb)

        pltpu.sync_copy(vals_ref.at[pl.ds(base, _FCH)], valb)

        def blk(j, cnt):
            key = keyb[pl.ds(j * 16, 16)]
            sl = plsc.load_gather(invb, [key])
            ot = othb[pl.ds(j * 16, 16)]
            vv = valb[pl.ds(j * 16, 16)]
            m = sl >= 0
            mi = jnp.where(m, jnp.full((16,), 1, jnp.int32), jnp.full((16,), 0, jnp.int32))
            csum = plsc.cumsum(mi)
            pos = jnp.where(m, cnt + csum - mi, _CAP)
            plsc.store_scatter(cbufA, [pos], sl)
            plsc.store_scatter(cbufB, [pos], ot)
            plsc.store_scatter(cbufV, [pos], vv)
            return cnt + csum[15]

        return lax.fori_loop(0, _FCH // 16, blk, cnt)

    cnt = lax.fori_loop(0, epw // _FCH, chunk, jnp.int32(0))
    pltpu.sync_copy(cbufA.at[pl.ds(0, _CAP)], segA_ref.at[c].at[s])
    pltpu.sync_copy(cbufB.at[pl.ds(0, _CAP)], segB_ref.at[c].at[s])
    pltpu.sync_copy(cbufV.at[pl.ds(0, _CAP)], segV_ref.at[c].at[s])
    cntb[pl.ds(0, 16)] = jnp.full((16,), cnt, jnp.int32)
    pltpu.sync_copy(cntb, cnt_ref.at[c].at[s])


@jax.jit
def _filter_sc(rows, cols, vals, uidx, ar256):
    mesh = plsc.VectorSubcoreMesh(core_axis_name="c", subcore_axis_name="s",
                                  num_cores=2, num_subcores=_NSUB)
    return pl.kernel(
        _filter_body,
        out_type=(jax.ShapeDtypeStruct((_N,), jnp.int32),
                  jax.ShapeDtypeStruct((2, _NSUB, _CAP), jnp.int32),
                  jax.ShapeDtypeStruct((2, _NSUB, _CAP), jnp.int32),
                  jax.ShapeDtypeStruct((2, _NSUB, _CAP), jnp.float32),
                  jax.ShapeDtypeStruct((2, _NSUB, 16), jnp.int32)),
        mesh=mesh,
        compiler_params=pltpu.CompilerParams(use_tc_tiling_on_sc=False, needs_layout_passes=False),
        scratch_types=[
            pltpu.VMEM((_N,), jnp.int32),
            pltpu.VMEM((256,), jnp.int32),
            pltpu.VMEM((256,), jnp.int32),
            pltpu.VMEM((_FCH,), jnp.int32),
            pltpu.VMEM((_FCH,), jnp.int32),
            pltpu.VMEM((_FCH,), jnp.float32),
            pltpu.VMEM((_CAP + 16,), jnp.int32),
            pltpu.VMEM((_CAP + 16,), jnp.int32),
            pltpu.VMEM((_CAP + 16,), jnp.float32),
            pltpu.VMEM((16,), jnp.int32),
            pltpu.SemaphoreType.DMA,
        ],
    )(rows, cols, vals, uidx, ar256)


def _newden_body(h_ref, d_ref, segA_ref, segB_ref, segV_ref, cnt_ref,
                 uidx_ref, ar256_ref, nd_ref, hu_ref,
                 accS0, accS1,
                 uv, arv, hu0, hu1, accL0, accL1,
                 slotb, colb, valb, gh0, gh1, gd0, gd1, cntb, sem):
    s = lax.axis_index("s")

    def zr(j, cc):
        accL0[j, pl.ds(0, 16)] = jnp.zeros((16,), jnp.float32)
        accL0[j, pl.ds(16, 16)] = jnp.zeros((16,), jnp.float32)
        accL1[j, pl.ds(0, 16)] = jnp.zeros((16,), jnp.float32)
        accL1[j, pl.ds(16, 16)] = jnp.zeros((16,), jnp.float32)
        return cc
    lax.fori_loop(0, 256, zr, 0)

    @pl.when(s == 0)
    def _zero_shared():
        pltpu.sync_copy(accL0, accS0)
        pltpu.sync_copy(accL1, accS1)

    pltpu.sync_copy(ar256_ref, arv)
    pltpu.sync_copy(uidx_ref, uv)
    pltpu.async_copy(h_ref.at[0].at[uv], hu0, sem).wait()
    pltpu.async_copy(h_ref.at[1].at[uv], hu1, sem).wait()

    pltpu.sync_copy(cnt_ref.at[0].at[s], cntb)
    n = cntb[pl.ds(0, 16)][0]
    plsc.subcore_barrier()

    def chunk(i, cc):
        base = i * _MCH
        pltpu.sync_copy(segA_ref.at[0].at[s].at[pl.ds(base, _MCH)], slotb)
        pltpu.sync_copy(segB_ref.at[0].at[s].at[pl.ds(base, _MCH)], colb)
        pltpu.sync_copy(segV_ref.at[0].at[s].at[pl.ds(base, _MCH)], valb)

        def san(j, cc2):
            e16 = base + j * 16 + lax.iota(jnp.int32, 16)
            live = e16 < n
            slotb[pl.ds(j * 16, 16)] = jnp.where(live, slotb[pl.ds(j * 16, 16)], 0)
            colb[pl.ds(j * 16, 16)] = jnp.where(live, colb[pl.ds(j * 16, 16)], 0)
            valb[pl.ds(j * 16, 16)] = jnp.where(live, valb[pl.ds(j * 16, 16)], 0.0)
            return cc2
        lax.fori_loop(0, _MCH // 16, san, 0)

        pltpu.async_copy(h_ref.at[0].at[colb], gh0, sem).wait()
        pltpu.async_copy(h_ref.at[1].at[colb], gh1, sem).wait()
        pltpu.async_copy(d_ref.at[0].at[colb], gd0, sem).wait()
        pltpu.async_copy(d_ref.at[1].at[colb], gd1, sem).wait()

        def blk(j, cc2):
            sl16 = slotb[pl.ds(j * 16, 16)]
            vv16 = valb[pl.ds(j * 16, 16)]
            for t in range(16):
                e = j * 16 + t
                p = sl16[t]
                v = vv16[t]
                part = (gh0[e, pl.ds(0, 16)] * hu0[p, pl.ds(0, 16)]
                        + gh0[e, pl.ds(16, 16)] * hu0[p, pl.ds(16, 16)]
                        + gh1[e, pl.ds(0, 16)] * hu1[p, pl.ds(0, 16)]
                        + gh1[e, pl.ds(16, 16)] * hu1[p, pl.ds(16, 16)])
                dot = jnp.sum(part)
                sigv = 1.0 / (1.0 + jnp.exp(jnp.full((16,), -dot, jnp.float32)))
                wv = sigv * v
                accL0[p, pl.ds(0, 16)] = accL0[p, pl.ds(0, 16)] + wv * gd0[e, pl.ds(0, 16)]
                accL0[p, pl.ds(16, 16)] = accL0[p, pl.ds(16, 16)] + wv * gd0[e, pl.ds(16, 16)]
                accL1[p, pl.ds(0, 16)] = accL1[p, pl.ds(0, 16)] + wv * gd1[e, pl.ds(0, 16)]
                accL1[p, pl.ds(16, 16)] = accL1[p, pl.ds(16, 16)] + wv * gd1[e, pl.ds(16, 16)]
            return cc2
        lax.fori_loop(0, _MCH // 16, blk, 0)
        return cc

    nch = lax.div(n + (_MCH - 1), _MCH)
    lax.fori_loop(0, nch, chunk, 0)
    pltpu.sync_copy(accL0, accS0.at[arv], add=True)
    pltpu.sync_copy(accL1, accS1.at[arv], add=True)
    plsc.subcore_barrier()

    @pl.when(s == 0)
    def _writeout():
        pltpu.sync_copy(accS0, nd_ref.at[0])
        pltpu.sync_copy(accS1, nd_ref.at[1])
        pltpu.sync_copy(hu0, hu_ref.at[0])
        pltpu.sync_copy(hu1, hu_ref.at[1])


@jax.jit
def _newden_sc(h_split, d_split, segA, segB, segV, counts, uidx, ar256):
    mesh = plsc.VectorSubcoreMesh(core_axis_name="c", subcore_axis_name="s",
                                  num_cores=2, num_subcores=_NSUB)
    return pl.kernel(
        _newden_body,
        out_type=(jax.ShapeDtypeStruct((2, 256, 32), jnp.float32),
                  jax.ShapeDtypeStruct((2, 256, 32), jnp.float32)),
        mesh=mesh,
        compiler_params=pltpu.CompilerParams(use_tc_tiling_on_sc=False, needs_layout_passes=False),
        scratch_types=[
            pltpu.VMEM_SHARED((256, 32), jnp.float32),
            pltpu.VMEM_SHARED((256, 32), jnp.float32),
            pltpu.VMEM((256,), jnp.int32),
            pltpu.VMEM((256,), jnp.int32),
            pltpu.VMEM((256, 32), jnp.float32),
            pltpu.VMEM((256, 32), jnp.float32),
            pltpu.VMEM((256, 32), jnp.float32),
            pltpu.VMEM((256, 32), jnp.float32),
            pltpu.VMEM((_MCH,), jnp.int32),
            pltpu.VMEM((_MCH,), jnp.int32),
            pltpu.VMEM((_MCH,), jnp.float32),
            pltpu.VMEM((_MCH, 32), jnp.float32),
            pltpu.VMEM((_MCH, 32), jnp.float32),
            pltpu.VMEM((_MCH, 32), jnp.float32),
            pltpu.VMEM((_MCH, 32), jnp.float32),
            pltpu.VMEM((16,), jnp.int32),
            pltpu.SemaphoreType.DMA,
        ],
    )(h_split, d_split, segA, segB, segV, counts, uidx, ar256)


def _dspmm_body(base2_ref, segA_ref, segB_ref, segV_ref, cnt_ref,
                nd0_ref, e0_ref, e1_ref, uidx_ref, out_ref,
                accum, uv, deltab, t0b, t1b, rowb, slotb, valb, contrib,
                cntb, sem):
    c = lax.axis_index("c")
    s = lax.axis_index("s")
    npw = _N // _NSUB
    pltpu.sync_copy(base2_ref.at[c].at[pl.ds(s * npw, npw)],
                    accum.at[pl.ds(s * npw, npw)])

    pltpu.sync_copy(uidx_ref, uv)
    pltpu.async_copy(e0_ref.at[c].at[uv], t0b, sem).wait()
    pltpu.async_copy(e1_ref.at[c].at[uv], t1b, sem).wait()
    pltpu.sync_copy(nd0_ref.at[c], deltab)

    def dl(j, cc):
        deltab[j, pl.ds(0, 16)] = deltab[j, pl.ds(0, 16)] - (t1b[j, pl.ds(0, 16)] - t0b[j, pl.ds(0, 16)])
        deltab[j, pl.ds(16, 16)] = deltab[j, pl.ds(16, 16)] - (t1b[j, pl.ds(16, 16)] - t0b[j, pl.ds(16, 16)])
        return cc
    lax.fori_loop(0, 256, dl, 0)

    pltpu.sync_copy(cnt_ref.at[1].at[s], cntb)
    n = cntb[pl.ds(0, 16)][0]
    plsc.subcore_barrier()

    def chunk(i, cc):
      base = i * _MCH

      @pl.when(base < n)
      def _live_chunk():
        pltpu.sync_copy(segA_ref.at[1].at[s].at[pl.ds(base, _MCH)], slotb)
        pltpu.sync_copy(segB_ref.at[1].at[s].at[pl.ds(base, _MCH)], rowb)
        pltpu.sync_copy(segV_ref.at[1].at[s].at[pl.ds(base, _MCH)], valb)

        def san(j, cc2):
            e16 = base + j * 16 + lax.iota(jnp.int32, 16)
            live = e16 < n
            slotb[pl.ds(j * 16, 16)] = jnp.where(live, slotb[pl.ds(j * 16, 16)], 0)
            rowb[pl.ds(j * 16, 16)] = jnp.where(live, rowb[pl.ds(j * 16, 16)], 0)
            valb[pl.ds(j * 16, 16)] = jnp.where(live, valb[pl.ds(j * 16, 16)], 0.0)
            return cc2
        lax.fori_loop(0, _MCH // 16, san, 0)

        def blk(j, cc2):
            sl16 = slotb[pl.ds(j * 16, 16)]
            vv16 = valb[pl.ds(j * 16, 16)]
            for t in range(16):
                e = j * 16 + t
                p = sl16[t]
                v = vv16[t]
                contrib[e, pl.ds(0, 16)] = v * deltab[p, pl.ds(0, 16)]
                contrib[e, pl.ds(16, 16)] = v * deltab[p, pl.ds(16, 16)]
            return cc2
        lax.fori_loop(0, _MCH // 16, blk, 0)
        pltpu.sync_copy(contrib, accum.at[rowb], add=True)
      return cc

    lax.fori_loop(0, _CAP // _MCH, chunk, 0)
    plsc.subcore_barrier()

    pltpu.sync_copy(accum.at[pl.ds(s * npw, npw)],
                    out_ref.at[c].at[pl.ds(s * npw, npw)])


@jax.jit
def _dspmm_sc(base2s, segA, segB, segV, counts, nd0, e0s, e1s, uidx):
    mesh = plsc.VectorSubcoreMesh(core_axis_name="c", subcore_axis_name="s",
                                  num_cores=2, num_subcores=_NSUB)
    return pl.kernel(
        _dspmm_body,
        out_type=jax.ShapeDtypeStruct((2, _N, 32), jnp.float32),
        mesh=mesh,
        compiler_params=pltpu.CompilerParams(use_tc_tiling_on_sc=False, needs_layout_passes=False),
        scratch_types=[
            pltpu.VMEM_SHARED((_N, 32), jnp.float32),
            pltpu.VMEM((256,), jnp.int32),
            pltpu.VMEM((256, 32), jnp.float32),
            pltpu.VMEM((256, 32), jnp.float32),
            pltpu.VMEM((256, 32), jnp.float32),
            pltpu.VMEM((_MCH,), jnp.int32),
            pltpu.VMEM((_MCH,), jnp.int32),
            pltpu.VMEM((_MCH,), jnp.float32),
            pltpu.VMEM((_MCH, 32), jnp.float32),
            pltpu.VMEM((16,), jnp.int32),
            pltpu.SemaphoreType.DMA,
        ],
    )(base2s, segA, segB, segV, counts, nd0, e0s, e1s, uidx)


def _final_mix_kernel(e0_ref, e1_ref, e2_ref, e3_ref, noise_ref, sh_ref,
                      wm_ref, ws_ref, wd_ref,
                      main_ref, mean_ref, std_ref, dec_ref,
                      h0_ref, h1_ref, h2_ref, b2_ref, t0_ref, s_ref):
    i = pl.program_id(0)
    e0 = e0_ref[...]
    e1 = e1_ref[...]
    e2 = e2_ref[...]
    e3 = e3_ref[...]
    m = (e0 + e1 + e2 + e3) * 0.25
    main_ref[...] = m
    mean = jnp.maximum(m @ wm_ref[...], 0.0)
    std = jnp.maximum(m @ ws_ref[...], 0.0)
    mean_ref[...] = mean
    std_ref[...] = std
    x = noise_ref[...] * std + mean
    dec = jnp.maximum(x @ wd_ref[...], 0.0)
    dec_ref[...] = dec
    sh = sh_ref[...]
    h0_ref[...] = jax.nn.sigmoid(sh * e0)
    h1_ref[...] = jax.nn.sigmoid(sh * e1)
    h2_ref[...] = jax.nn.sigmoid(sh * e2)
    b2_ref[...] = e2 - 2.0 * e1 + e0

    @pl.when(i == 0)
    def _init():
        t0_ref[...] = jnp.zeros((64, 64), jnp.float32)
        s_ref[...] = jnp.zeros((64, 64), jnp.float32)

    t0_ref[...] += dec.T @ e0
    s_ref[...] += dec.T @ dec


def _dense_pass(e0, e1, e2, e3, noise, shared, W_mean, W_std, W_decoder):
    n = e0.shape[0]
    blk = 3200
    grid = (n // blk,)
    bs = pl.BlockSpec((blk, 64), lambda i: (i, 0))
    ws = pl.BlockSpec((64, 64), lambda i: (0, 0))
    shs = pl.BlockSpec((1, 64), lambda i: (0, 0))
    acc = pl.BlockSpec((64, 64), lambda i: (0, 0))
    out_sd = jax.ShapeDtypeStruct((n, 64), jnp.float32)
    w_sd = jax.ShapeDtypeStruct((64, 64), jnp.float32)
    return pl.pallas_call(
        _final_mix_kernel,
        grid=grid,
        in_specs=[bs, bs, bs, bs, bs, shs, ws, ws, ws],
        out_specs=[bs, bs, bs, bs, bs, bs, bs, bs, acc, acc],
        out_shape=[out_sd] * 8 + [w_sd, w_sd],
    )(e0, e1, e2, e3, noise, shared, W_mean, W_std, W_decoder)


def _m_kernel(t0_ref, s_ref, m_ref):
    t0 = t0_ref[...]
    s = s_ref[...]
    st = s @ t0
    m_ref[...] = t0 + st + s @ st


def _gen_m_pass(t0, S):
    return pl.pallas_call(
        _m_kernel,
        out_shape=jax.ShapeDtypeStruct((64, 64), jnp.float32),
    )(t0, S)


def _ov_kernel(base_ref, nd_ref, inv_ref, out_ref):
    inv = inv_ref[...]
    flag = inv >= 0
    oh = (inv == lax.broadcasted_iota(jnp.int32, (1, 256), 1)).astype(jnp.float32)
    out_ref[...] = jnp.where(flag, oh @ nd_ref[...], base_ref[...])


def _ov_pass(base, nd, inv2d):
    n = base.shape[0]
    blk = 3200
    bs = pl.BlockSpec((blk, 64), lambda i: (i, 0))
    return pl.pallas_call(
        _ov_kernel,
        grid=(n // blk,),
        in_specs=[bs, pl.BlockSpec((256, 64), lambda i: (0, 0)),
                  pl.BlockSpec((blk, 1), lambda i: (i, 0))],
        out_specs=bs,
        out_shape=jax.ShapeDtypeStruct((n, 64), jnp.float32),
    )(base, nd, inv2d)


def _fin_kernel(e0_ref, d1_ref, d2_ref, d3r_ref, nd2_ref, inv_ref,
                dec_ref, m_ref, h0_ref, h1_ref, h2_ref,
                hu0_ref, hu1_ref, hu2_ref,
                den_ref, gen_ref, dg0_ref, dg1_ref, dg2_ref):
    inv = inv_ref[...]
    flag = inv >= 0
    oh = (inv == lax.broadcasted_iota(jnp.int32, (1, 256), 1)).astype(jnp.float32)
    d3 = jnp.where(flag, oh @ nd2_ref[...], d3r_ref[...])
    e0 = e0_ref[...]
    den_ref[...] = (e0 + d1_ref[...] + d2_ref[...] + d3) * 0.25
    gen_ref[...] = (e0 + dec_ref[...] @ m_ref[...]) * 0.25
    dg0_ref[...] = jax.nn.sigmoid(hu0_ref[...] @ h0_ref[...].T)
    dg1_ref[...] = jax.nn.sigmoid(hu1_ref[...] @ h1_ref[...].T)
    dg2_ref[...] = jax.nn.sigmoid(hu2_ref[...] @ h2_ref[...].T)


def _final_pass(e0, d1, d2, d3raw, nd2, inv2d, dec, M, h0, h1, h2,
                hu0, hu1, hu2):
    n = e0.shape[0]
    blk = 3200
    bs = pl.BlockSpec((blk, 64), lambda i: (i, 0))
    hus = pl.BlockSpec((256, 64), lambda i: (0, 0))
    ms = pl.BlockSpec((64, 64), lambda i: (0, 0))
    ivs = pl.BlockSpec((blk, 1), lambda i: (i, 0))
    dgs = pl.BlockSpec((256, blk), lambda i: (0, i))
    out_sd = jax.ShapeDtypeStruct((n, 64), jnp.float32)
    dg_sd = jax.ShapeDtypeStruct((256, n), jnp.float32)
    return pl.pallas_call(
        _fin_kernel,
        grid=(n // blk,),
        in_specs=[bs, bs, bs, bs, hus, ivs, bs, ms, bs, bs, bs,
                  hus, hus, hus],
        out_specs=[bs, bs, dgs, dgs, dgs],
        out_shape=[out_sd, out_sd, dg_sd, dg_sd, dg_sd],
    )(e0, d1, d2, d3raw, nd2, inv2d, dec, M, h0, h1, h2, hu0, hu1, hu2)


def kernel(user_emb, item_emb, W_mean, W_std, W_decoder, shared_layer,
           adj_rows, adj_cols, adj_vals, user_idx, noise):
    user_num = user_emb.shape[0]
    e0 = jnp.concatenate([user_emb, item_emb], axis=0)
    n_total = e0.shape[0]

    zeros_split = jnp.zeros((2, n_total, 32), jnp.float32)
    e0s = _to_split(e0)
    e1s = _spmm_sc(e0s, e0s, adj_cols, adj_rows, adj_vals)
    e2s = _spmm_sc(e1s, e1s, adj_cols, adj_rows, adj_vals)
    e3s = _spmm_sc(e2s, e2s, adj_cols, adj_rows, adj_vals)
    e1 = _from_split(e1s)
    e2 = _from_split(e2s)
    e3 = _from_split(e3s)

    (main_all, mean, std, D, h0, h1, h2, base2, t0, S) = _dense_pass(
        e0, e1, e2, e3, noise, shared_layer, W_mean, W_std, W_decoder)
    M = _gen_m_pass(t0, S)

    ar256 = jnp.arange(256, dtype=jnp.int32)
    inv, segA, segB, segV, counts = _filter_sc(adj_rows, adj_cols, adj_vals,
                                               user_idx, ar256)
    inv2d = inv[:, None]
    slot_of = inv[user_idx]

    def _dup(nds):
        full = jnp.concatenate([nds[0], nds[1]], axis=1)[slot_of]
        return full, jnp.stack([full[:, :32], full[:, 32:]], axis=0)

    hs = [_to_split(hk) for hk in (h0, h1, h2)]

    nd0r, hu0 = _newden_sc(hs[0], e0s, segA, segB, segV, counts, user_idx, ar256)
    nd0, nd0s = _dup(nd0r)
    d1 = _ov_pass(e1 - e0, nd0, inv2d)

    d1s = _to_split(d1)
    nd1r, hu1 = _newden_sc(hs[1], d1s, segA, segB, segV, counts, user_idx, ar256)
    nd1, _ = _dup(nd1r)
    d2raw = _from_split(_dspmm_sc(_to_split(base2), segA, segB, segV, counts,
                                  nd0s, e0s, e1s, user_idx))
    d2 = _ov_pass(d2raw, nd1, inv2d)
    d2s = _to_split(d2)
    nd2r, hu2 = _newden_sc(hs[2], d2s, segA, segB, segV, counts, user_idx, ar256)
    nd2, _ = _dup(nd2r)
    d3raw = _from_split(_spmm_sc(d2s, zeros_split, adj_cols, adj_rows, adj_vals))

    def _huf(hu):
        return jnp.concatenate([hu[0], hu[1]], axis=1)

    den_m, gen_m, dg0, dg1, dg2 = _final_pass(
        e0, d1, d2, d3raw, nd2, inv2d, D, M, h0, h1, h2,
        _huf(hu0), _huf(hu1), _huf(hu2))

    return (main_all[:user_num], main_all[user_num:],
            gen_m[:user_num], gen_m[user_num:],
            den_m[:user_num], den_m[user_num:],
            mean, std, (dg0, dg1, dg2))
